# Initial kernel scaffold; baseline (speedup 1.0000x reference)
import jax
import jax.numpy as jnp
from jax.experimental import pallas as pl


def kernel(X):
    raise NotImplementedError("write your pallas kernel here")


# speedup vs baseline: 5.5097x; 5.5097x over previous
import jax
import jax.numpy as jnp
from jax import lax
from jax.experimental import pallas as pl
from jax.experimental.pallas import tpu as pltpu
from jax.experimental.pallas import tpu_sc as plsc

N_ROWS = 128
N_COLS = 32768
RANK_IDX = int(0.9 * N_COLS)
K_TOP = N_COLS - RANK_IDX
LANES = 16
CHUNKS = N_COLS // LANES
NC, NS = 2, 16
NW = NC * NS
ROWS_PER_W = N_ROWS // NW

_MESH = plsc.VectorSubcoreMesh(
    core_axis_name="c", subcore_axis_name="s", num_cores=NC, num_subcores=NS
)


def _tec_body(x_hbm, out_hbm, xbuf, kbuf, hist):
    cid = lax.axis_index("c")
    sid = lax.axis_index("s")
    wid = sid * NC + cid

    ones_i32 = jnp.full((LANES,), 1, jnp.int32)
    zeros_i32 = jnp.zeros((LANES,), jnp.int32)

    def do_row(r, _):
        row = wid * ROWS_PER_W + r
        pltpu.sync_copy(x_hbm.at[row], xbuf)

        for j in range(16):
            hist[pl.ds(j * LANES, LANES)] = zeros_i32

        def l0(i, carry):
            x16 = xbuf[pl.ds(i * LANES, LANES)]
            xu = plsc.bitcast(x16, jnp.uint32)
            sign = xu >> 31
            key = xu ^ ((jnp.uint32(0) - sign) | jnp.uint32(0x80000000))
            kbuf[pl.ds(i * LANES, LANES)] = key
            b = (key >> 24).astype(jnp.int32)
            plsc.addupdate_scatter(hist, [b], ones_i32)
            return carry

        lax.fori_loop(0, CHUNKS, l0, 0)

        def pick_byte(r_cur):
            def scan(jj, carry):
                above, nge, cab = carry
                j = 15 - jj
                v = hist[pl.ds(j * LANES, LANES)]
                cs = jnp.cumsum(lax.rev(v, (0,)))
                C = lax.rev(cs, (0,)) + above
                ge = C >= r_cur
                nge = nge + jnp.sum(ge.astype(jnp.int32))
                cab = jnp.maximum(cab, jnp.max(jnp.where(ge, 0, C)))
                return (above + jnp.sum(v), nge, cab)

            _, nge, c_above = lax.fori_loop(
                0, 16, scan, (jnp.int32(0), jnp.int32(0), jnp.int32(0))
            )
            return nge - 1, r_cur - c_above

        r_cur = jnp.int32(K_TOP)
        B, r_cur = pick_byte(r_cur)
        prefix = B.astype(jnp.uint32)

        for lvl in range(1, 4):
            s = 24 - 8 * lvl
            for j in range(16):
                hist[pl.ds(j * LANES, LANES)] = zeros_i32

            def lv(i, carry, s=s, prefix=prefix):
                key = kbuf[pl.ds(i * LANES, LANES)]
                match = (key >> (s + 8)) == prefix
                b = ((key >> s) & 0xFF).astype(jnp.int32)
                plsc.addupdate_scatter(hist, [b], ones_i32, mask=match)
                return carry

            lax.fori_loop(0, CHUNKS, lv, 0)
            B, r_cur = pick_byte(r_cur)
            prefix = prefix * jnp.uint32(256) + B.astype(jnp.uint32)

        kthr = prefix

        def mk(i, carry):
            key = kbuf[pl.ds(i * LANES, LANES)]
            x16 = xbuf[pl.ds(i * LANES, LANES)]
            xbuf[pl.ds(i * LANES, LANES)] = jnp.where(key >= kthr, x16, 0.0)
            return carry

        lax.fori_loop(0, CHUNKS, mk, 0)
        pltpu.sync_copy(xbuf, out_hbm.at[row])
        return _

    lax.fori_loop(0, ROWS_PER_W, do_row, 0)


_ksparse_sc = pl.kernel(
    _tec_body,
    out_type=jax.ShapeDtypeStruct((N_ROWS, N_COLS), jnp.float32),
    mesh=_MESH,
    scratch_types=[
        pltpu.VMEM((N_COLS,), jnp.float32),
        pltpu.VMEM((N_COLS,), jnp.uint32),
        pltpu.VMEM((256,), jnp.int32),
    ],
    compiler_params=pltpu.CompilerParams(needs_layout_passes=False),
    name="ksparse_radix_select_sc",
)


def kernel(X):
    return _ksparse_sc(X)


# speedup vs baseline: 6.9534x; 1.2620x over previous
import jax
import jax.numpy as jnp
from jax import lax
from jax.experimental import pallas as pl
from jax.experimental.pallas import tpu as pltpu
from jax.experimental.pallas import tpu_sc as plsc

N_ROWS = 128
N_COLS = 32768
RANK_IDX = int(0.9 * N_COLS)
K_TOP = N_COLS - RANK_IDX
LANES = 16
CHUNKS = N_COLS // LANES
NC, NS = 2, 16
NW = NC * NS
ROWS_PER_W = N_ROWS // NW
U = 2

_MESH = plsc.VectorSubcoreMesh(
    core_axis_name="c", subcore_axis_name="s", num_cores=NC, num_subcores=NS
)

_MININT = -2147483648


def _keys_of(x16):
    xi = plsc.bitcast(x16, jnp.int32)
    m = lax.shift_right_arithmetic(xi, 31)
    return xi ^ (m | _MININT)


def _srl(x, n):
    return lax.shift_right_logical(x, n)


def _tec_body(x_hbm, out_hbm, xbuf, cbuf, hist):
    cid = lax.axis_index("c")
    sid = lax.axis_index("s")
    wid = sid * NC + cid

    ones = jnp.full((LANES,), 1, jnp.int32)
    zeros = jnp.zeros((LANES,), jnp.int32)
    iota = lax.iota(jnp.int32, LANES)

    def zero_hist():
        for j in range(256 // LANES):
            hist[pl.ds(j * LANES, LANES)] = zeros

    def pick_byte(r_cur):
        def scan(jj, carry):
            above, nge, cab = carry
            j = 15 - jj
            v = hist[pl.ds(j * LANES, LANES)]
            cs = jnp.cumsum(lax.rev(v, (0,)))
            C = lax.rev(cs, (0,)) + above
            ge = C >= r_cur
            nge = nge + jnp.sum(ge.astype(jnp.int32))
            cab = jnp.maximum(cab, jnp.max(jnp.where(ge, 0, C)))
            return (above + jnp.sum(v), nge, cab)

        _, nge, c_above = lax.fori_loop(
            0, 16, scan, (jnp.int32(0), jnp.int32(0), jnp.int32(0))
        )
        return nge - 1, r_cur - c_above

    def do_row(row, _):
        pltpu.sync_copy(x_hbm.at[row], xbuf)

        zero_hist()

        def l0(i, carry):
            for u in range(U):
                key = _keys_of(xbuf[pl.ds((U * i + u) * LANES, LANES)])
                b = _srl(key, 24)
                plsc.addupdate_scatter(hist, [b], ones)
            return carry

        lax.fori_loop(0, CHUNKS // U, l0, 0)
        B, r_cur = pick_byte(jnp.int32(K_TOP))
        p8 = B

        zero_hist()

        def l1(i, w):
            for u in range(U):
                key = _keys_of(xbuf[pl.ds((U * i + u) * LANES, LANES)])
                match = _srl(key, 24) == p8
                b = _srl(key, 16) & 0xFF
                plsc.addupdate_scatter(hist, [b], ones, mask=match)
                pos = w + jnp.cumsum(match.astype(jnp.int32)) - 1
                plsc.store_scatter(cbuf, [pos], key, mask=match)
                w = w + plsc.all_reduce_population_count(match)
            return w

        n1v = lax.fori_loop(0, CHUNKS // U, l1, zeros)
        n1 = jnp.max(n1v)
        B, r_cur = pick_byte(r_cur)
        p16 = (p8 << 8) | B

        zero_hist()

        def l2(i, w):
            base = i * LANES
            key = cbuf[pl.ds(base, LANES)]
            match = (_srl(key, 16) == p16) & (iota < (n1 - base))
            b = _srl(key, 8) & 0xFF
            plsc.addupdate_scatter(hist, [b], ones, mask=match)
            pos = w + jnp.cumsum(match.astype(jnp.int32)) - 1
            plsc.store_scatter(cbuf, [pos], key, mask=match)
            return w + plsc.all_reduce_population_count(match)

        n2v = lax.fori_loop(0, (n1 + LANES - 1) // LANES, l2, zeros)
        n2 = jnp.max(n2v)
        B, r_cur = pick_byte(r_cur)
        p24 = (p16 << 8) | B

        zero_hist()

        def l3(i, carry):
            base = i * LANES
            key = cbuf[pl.ds(base, LANES)]
            match = (_srl(key, 8) == p24) & (iota < (n2 - base))
            b = key & 0xFF
            plsc.addupdate_scatter(hist, [b], ones, mask=match)
            return carry

        lax.fori_loop(0, (n2 + LANES - 1) // LANES, l3, 0)
        B, _ = pick_byte(r_cur)
        kthr = lax.shift_left(p24, 8) | B

        kv = jnp.broadcast_to(kthr, (LANES,))
        tb = jnp.where(kv < 0, kv ^ _MININT, ~kv)
        theta = plsc.bitcast(tb, jnp.float32)

        def mk(i, carry):
            for u in range(U):
                sl = pl.ds((U * i + u) * LANES, LANES)
                x16 = xbuf[sl]
                xbuf[sl] = jnp.where(x16 >= theta, x16, 0.0)
            return carry

        lax.fori_loop(0, CHUNKS // U, mk, 0)
        pltpu.sync_copy(xbuf, out_hbm.at[row])
        return _

    lax.fori_loop(wid * ROWS_PER_W, (wid + 1) * ROWS_PER_W, do_row, 0)


_ksparse_sc = pl.kernel(
    _tec_body,
    out_type=jax.ShapeDtypeStruct((N_ROWS, N_COLS), jnp.float32),
    mesh=_MESH,
    scratch_types=[
        pltpu.VMEM((N_COLS,), jnp.float32),
        pltpu.VMEM((N_COLS,), jnp.int32),
        pltpu.VMEM((256,), jnp.int32),
    ],
    compiler_params=pltpu.CompilerParams(needs_layout_passes=False),
    name="ksparse_radix_select_sc",
)


def kernel(X):
    return _ksparse_sc(X)


# speedup vs baseline: 15.8009x; 2.2724x over previous
import jax
import jax.numpy as jnp
from jax import lax
from jax.experimental import pallas as pl
from jax.experimental.pallas import tpu as pltpu
from jax.experimental.pallas import tpu_sc as plsc

N_ROWS = 128
N_COLS = 32768
RANK_IDX = int(0.9 * N_COLS)
K_TOP = N_COLS - RANK_IDX
LANES = 16
CHUNKS = N_COLS // LANES
NC, NS = 2, 16
NW = NC * NS
ROWS_PER_W = N_ROWS // NW
U = 2

_MESH = plsc.VectorSubcoreMesh(
    core_axis_name="c", subcore_axis_name="s", num_cores=NC, num_subcores=NS
)

_MININT = -2147483648


def _keys_of(x16):
    xi = plsc.bitcast(x16, jnp.int32)
    m = lax.shift_right_arithmetic(xi, 31)
    return xi ^ (m | _MININT)


def _srl(x, n):
    return lax.shift_right_logical(x, n)


def _tec_body(x_hbm, out_hbm, xbuf, cbuf, hist):
    cid = lax.axis_index("c")
    sid = lax.axis_index("s")
    wid = sid * NC + cid

    ones = jnp.full((LANES,), 1, jnp.int32)
    zeros = jnp.zeros((LANES,), jnp.int32)
    iota = lax.iota(jnp.int32, LANES)

    def zero_hist():
        for j in range(256 // LANES):
            hist[pl.ds(j * LANES, LANES)] = zeros

    def pick_byte(r_cur):
        above = jnp.int32(0)
        nge = jnp.int32(0)
        cab = jnp.int32(0)
        for j in range(15, -1, -1):
            v = hist[pl.ds(j * LANES, LANES)]
            cs = jnp.cumsum(lax.rev(v, (0,)))
            C = lax.rev(cs, (0,)) + above
            ge = C >= r_cur
            nge = nge + jnp.sum(ge.astype(jnp.int32))
            cab = jnp.maximum(cab, jnp.max(jnp.where(ge, 0, C)))
            above = above + jnp.sum(v)
        return nge - 1, r_cur - cab

    def do_row(row, _):
        pltpu.sync_copy(x_hbm.at[row], xbuf)

        zero_hist()

        @plsc.parallel_loop(0, CHUNKS, unroll=8)
        def l0(i):
            key = _keys_of(xbuf[pl.ds(i * LANES, LANES)])
            b = _srl(key, 24)
            plsc.addupdate_scatter(hist, [b], ones)
        B, r_cur = pick_byte(jnp.int32(K_TOP))
        p8 = B

        zero_hist()

        @plsc.parallel_loop(0, CHUNKS, unroll=4, carry=zeros)
        def l1(i, w):
            key = _keys_of(xbuf[pl.ds(i * LANES, LANES)])
            match = _srl(key, 24) == p8
            b = _srl(key, 16) & 0xFF
            plsc.addupdate_scatter(hist, [b], ones, mask=match)
            pos = w + jnp.cumsum(match.astype(jnp.int32)) - 1
            plsc.store_scatter(cbuf, [pos], key, mask=match)
            return w + plsc.all_reduce_population_count(match)

        n1 = jnp.max(l1)
        B, r_cur = pick_byte(r_cur)
        p16 = (p8 << 8) | B

        zero_hist()

        def l2(i, w):
            base = i * LANES
            key = cbuf[pl.ds(base, LANES)]
            match = (_srl(key, 16) == p16) & (iota < (n1 - base))
            b = _srl(key, 8) & 0xFF
            plsc.addupdate_scatter(hist, [b], ones, mask=match)
            pos = w + jnp.cumsum(match.astype(jnp.int32)) - 1
            plsc.store_scatter(cbuf, [pos], key, mask=match)
            return w + plsc.all_reduce_population_count(match)

        n2v = lax.fori_loop(0, (n1 + LANES - 1) // LANES, l2, zeros)
        n2 = jnp.max(n2v)
        B, r_cur = pick_byte(r_cur)
        p24 = (p16 << 8) | B

        zero_hist()

        def l3(i, carry):
            base = i * LANES
            key = cbuf[pl.ds(base, LANES)]
            match = (_srl(key, 8) == p24) & (iota < (n2 - base))
            b = key & 0xFF
            plsc.addupdate_scatter(hist, [b], ones, mask=match)
            return carry

        lax.fori_loop(0, (n2 + LANES - 1) // LANES, l3, 0)
        B, _ = pick_byte(r_cur)
        kthr = lax.shift_left(p24, 8) | B

        kv = jnp.broadcast_to(kthr, (LANES,))
        tb = jnp.where(kv < 0, kv ^ _MININT, ~kv)
        theta = plsc.bitcast(tb, jnp.float32)

        @plsc.parallel_loop(0, CHUNKS, unroll=8)
        def mk(i):
            sl = pl.ds(i * LANES, LANES)
            x16 = xbuf[sl]
            xbuf[sl] = jnp.where(x16 >= theta, x16, 0.0)
        pltpu.sync_copy(xbuf, out_hbm.at[row])
        return _

    lax.fori_loop(wid * ROWS_PER_W, (wid + 1) * ROWS_PER_W, do_row, 0)


_ksparse_sc = pl.kernel(
    _tec_body,
    out_type=jax.ShapeDtypeStruct((N_ROWS, N_COLS), jnp.float32),
    mesh=_MESH,
    scratch_types=[
        pltpu.VMEM((N_COLS,), jnp.float32),
        pltpu.VMEM((N_COLS,), jnp.int32),
        pltpu.VMEM((256,), jnp.int32),
    ],
    compiler_params=pltpu.CompilerParams(needs_layout_passes=False),
    name="ksparse_radix_select_sc",
)


def kernel(X):
    return _ksparse_sc(X)
